# Initial kernel scaffold; baseline (speedup 1.0000x reference)
import jax
import jax.numpy as jnp
from jax.experimental import pallas as pl


def kernel(significance, weight, edge_index, embed_table, W_lin, b_lin, W_conv0, b_conv0, W_conv1, b_conv1, W_pred, b_pred):
    raise NotImplementedError("write your pallas kernel here")


# speedup vs baseline: 32.6535x; 32.6535x over previous
import functools

import jax
import jax.numpy as jnp
from jax import lax
from jax.experimental import pallas as pl
from jax.experimental.pallas import tpu as pltpu
from jax.experimental.pallas import tpu_sc as plsc

N_CORES = 2
N_SUBCORES = 16
N_WORKERS = N_CORES * N_SUBCORES
LANES = 16


def _zero_vmem(ref, npad):
    zero = jnp.zeros((LANES,), jnp.float32)

    def body(i, _):
        ref[pl.ds(i * LANES, LANES)] = zero
        return 0

    lax.fori_loop(0, npad // LANES, body, 0)


def _publish_reduce_emit(acc, shared, out_hbm, tmp, accr, sid, npad, out_base):
    stripe = npad // N_SUBCORES
    nv = stripe // LANES
    pltpu.sync_copy(acc, shared.at[pl.ds(sid * npad, npad)])
    plsc.subcore_barrier()
    st = sid * stripe
    pltpu.sync_copy(shared.at[pl.ds(st, stripe)], accr)
    for j in range(1, N_SUBCORES):
        pltpu.sync_copy(shared.at[pl.ds(j * npad + st, stripe)], tmp)

        def addk(k, _):
            accr[pl.ds(k * LANES, LANES)] = (
                accr[pl.ds(k * LANES, LANES)] + tmp[pl.ds(k * LANES, LANES)]
            )
            return 0

        lax.fori_loop(0, nv, addk, 0)
    pltpu.sync_copy(accr, out_hbm.at[pl.ds(out_base + st, stripe)])
    plsc.subcore_barrier()


def _make_degree_kernel(e, npad):
    ch = e // N_SUBCORES
    mesh = plsc.VectorSubcoreMesh(core_axis_name="c", subcore_axis_name="s")

    @functools.partial(
        pl.kernel,
        out_type=jax.ShapeDtypeStruct((2 * npad,), jnp.float32),
        mesh=mesh,
        compiler_params=pltpu.CompilerParams(needs_layout_passes=False),
        scratch_types=[
            pltpu.VMEM((ch,), jnp.int32),
            pltpu.VMEM((npad,), jnp.float32),
            pltpu.VMEM((npad // N_SUBCORES,), jnp.float32),
            pltpu.VMEM((npad // N_SUBCORES,), jnp.float32),
            pltpu.VMEM_SHARED((N_SUBCORES * npad,), jnp.float32),
        ],
    )
    def deg_kernel(edges_hbm, out_hbm, idx_v, cnt, tmp, accr, shared):
        cid = lax.axis_index("c")
        sid = lax.axis_index("s")
        pltpu.sync_copy(edges_hbm.at[pl.ds(cid * e + sid * ch, ch)], idx_v)
        _zero_vmem(cnt, npad)
        ones = jnp.full((LANES,), 1.0, jnp.float32)

        def body(i, _):
            ix = idx_v[pl.ds(i * LANES, LANES)]
            plsc.addupdate_scatter(cnt, [ix], ones)
            return 0

        lax.fori_loop(0, ch // LANES, body, 0)
        _publish_reduce_emit(cnt, shared, out_hbm, tmp, accr, sid, npad, cid * npad)

    return deg_kernel


def _make_agg_kernel(e, npad, n_chan):
    ch = e // N_WORKERS
    mesh = plsc.VectorSubcoreMesh(core_axis_name="c", subcore_axis_name="s")

    scratch = [
        pltpu.VMEM((ch,), jnp.int32),
        pltpu.VMEM((ch,), jnp.int32),
        pltpu.VMEM((npad // N_SUBCORES,), jnp.float32),
        pltpu.VMEM((npad // N_SUBCORES,), jnp.float32),
        pltpu.VMEM_SHARED((N_SUBCORES * npad,), jnp.float32),
    ]
    for _ in range(n_chan):
        scratch.append(pltpu.VMEM((npad,), jnp.float32))
    for _ in range(n_chan):
        scratch.append(pltpu.VMEM((npad,), jnp.float32))

    @functools.partial(
        pl.kernel,
        out_type=jax.ShapeDtypeStruct((2 * n_chan * npad,), jnp.float32),
        mesh=mesh,
        compiler_params=pltpu.CompilerParams(needs_layout_passes=False),
        scratch_types=scratch,
    )
    def agg_kernel(edges_hbm, tab_hbm, out_hbm, src_v, dst_v, tmp, accr, shared, *gs_as):
        gs = gs_as[:n_chan]
        accs = gs_as[n_chan:]
        cid = lax.axis_index("c")
        sid = lax.axis_index("s")
        base = cid * (e // 2) + sid * ch
        pltpu.sync_copy(edges_hbm.at[pl.ds(base, ch)], src_v)
        pltpu.sync_copy(edges_hbm.at[pl.ds(e + base, ch)], dst_v)
        for c in range(n_chan):
            pltpu.sync_copy(tab_hbm.at[pl.ds(c * npad, npad)], gs[c])
            _zero_vmem(accs[c], npad)

        def body(i, _):
            ix_s = src_v[pl.ds(i * LANES, LANES)]
            ix_d = dst_v[pl.ds(i * LANES, LANES)]
            for c in range(n_chan):
                v = plsc.load_gather(gs[c], [ix_s])
                plsc.addupdate_scatter(accs[c], [ix_d], v)
            return 0

        lax.fori_loop(0, ch // LANES, body, 0)
        for c in range(n_chan):
            _publish_reduce_emit(accs[c], shared, out_hbm, tmp, accr, sid, npad,
                                 (cid * n_chan + c) * npad)

    return agg_kernel


def _feat_kernel(deg_ref, w_ref, sig_ref, et_ref, g_ref, rin_ref):
    d_out = deg_ref[0:1, :]
    d_in = deg_ref[1:2, :]
    r_out = lax.rsqrt(d_out + 1.0)
    rin_ref[...] = lax.rsqrt(d_in + 1.0)
    is1 = sig_ref[...] == 1
    s0 = jnp.where(is1, et_ref[1, 0], et_ref[0, 0])
    s1 = jnp.where(is1, et_ref[1, 1], et_ref[0, 1])
    g_ref[0:1, :] = w_ref[...] * r_out
    g_ref[1:2, :] = s0 * r_out
    g_ref[2:3, :] = s1 * r_out
    g_ref[3:4, :] = r_out


def _dense_kernel(agg0_ref, agg1_ref, g_ref, rin_ref, wlin_ref, wc0_ref,
                  wc1_ref, wp_ref, blin_ref, bc0_ref, p_ref):
    a = (agg0_ref[...] + agg1_ref[...] + g_ref[...]) * rin_ref[...]
    a3 = a[0:3, :]
    s = a[3:4, :]
    dn = (((1,), (0,)), ((), ()))
    t1 = lax.dot_general(wlin_ref[...], a3, dn, preferred_element_type=jnp.float32, precision=lax.Precision.HIGHEST)
    t1 = t1 + blin_ref[...] * s
    emb = lax.dot_general(wc0_ref[...], t1, dn, preferred_element_type=jnp.float32, precision=lax.Precision.HIGHEST)
    emb = emb + bc0_ref[...]
    h = jnp.where(emb >= 0.0, emb, 0.01 * emb)
    t2 = lax.dot_general(wc1_ref[...], h, dn, preferred_element_type=jnp.float32, precision=lax.Precision.HIGHEST)
    pq = lax.dot_general(wp_ref[...], t2, dn, preferred_element_type=jnp.float32, precision=lax.Precision.HIGHEST)
    p_ref[...] = pq * g_ref[3:4, :]


def _final_kernel(t_ref, p_ref, rin_ref, wp_ref, bc1_ref, bp_ref, out_ref):
    c = jnp.sum(wp_ref[...] * bc1_ref[...]) + bp_ref[0, 0]
    out_ref[...] = (t_ref[0:1, :] + t_ref[1:2, :] + p_ref[...]) * rin_ref[...] + c


def kernel(significance, weight, edge_index, embed_table, W_lin, b_lin,
           W_conv0, b_conv0, W_conv1, b_conv1, W_pred, b_pred):
    n = significance.shape[0]
    e = edge_index.shape[1]
    d = W_lin.shape[0]
    npad = ((n + 2047) // 2048) * 2048

    edges = edge_index.astype(jnp.int32).reshape(-1)
    wpad = jnp.pad(weight.astype(jnp.float32), (0, npad - n)).reshape(1, npad)
    sigpad = jnp.pad(significance.astype(jnp.int32), (0, npad - n)).reshape(1, npad)

    deg = _make_degree_kernel(e, npad)(edges).reshape(2, npad)

    g, rin = pl.pallas_call(
        _feat_kernel,
        out_shape=(
            jax.ShapeDtypeStruct((4, npad), jnp.float32),
            jax.ShapeDtypeStruct((1, npad), jnp.float32),
        ),
        in_specs=[
            pl.BlockSpec((2, npad), lambda: (0, 0)),
            pl.BlockSpec((1, npad), lambda: (0, 0)),
            pl.BlockSpec((1, npad), lambda: (0, 0)),
            pl.BlockSpec(memory_space=pltpu.SMEM),
        ],
        out_specs=(
            pl.BlockSpec((4, npad), lambda: (0, 0)),
            pl.BlockSpec((1, npad), lambda: (0, 0)),
        ),
    )(deg, wpad, sigpad, embed_table.astype(jnp.float32))

    aggp = _make_agg_kernel(e, npad, 4)(edges, g.reshape(-1)).reshape(2, 4, npad)

    bw = 512
    grid = (npad // bw,)
    p = pl.pallas_call(
        _dense_kernel,
        grid=grid,
        out_shape=jax.ShapeDtypeStruct((1, npad), jnp.float32),
        in_specs=[
            pl.BlockSpec((4, bw), lambda i: (0, i)),
            pl.BlockSpec((4, bw), lambda i: (0, i)),
            pl.BlockSpec((4, bw), lambda i: (0, i)),
            pl.BlockSpec((1, bw), lambda i: (0, i)),
            pl.BlockSpec((d, 3), lambda i: (0, 0)),
            pl.BlockSpec((d, d), lambda i: (0, 0)),
            pl.BlockSpec((d, d), lambda i: (0, 0)),
            pl.BlockSpec((1, d), lambda i: (0, 0)),
            pl.BlockSpec((d, 1), lambda i: (0, 0)),
            pl.BlockSpec((d, 1), lambda i: (0, 0)),
        ],
        out_specs=pl.BlockSpec((1, bw), lambda i: (0, i)),
    )(
        aggp[0], aggp[1], g, rin,
        W_lin.astype(jnp.float32), W_conv0.astype(jnp.float32),
        W_conv1.astype(jnp.float32), W_pred.astype(jnp.float32),
        b_lin.astype(jnp.float32).reshape(d, 1),
        b_conv0.astype(jnp.float32).reshape(d, 1),
    )

    t = _make_agg_kernel(e, npad, 1)(edges, p.reshape(-1)).reshape(2, npad)

    out = pl.pallas_call(
        _final_kernel,
        out_shape=jax.ShapeDtypeStruct((1, npad), jnp.float32),
        in_specs=[
            pl.BlockSpec((2, npad), lambda: (0, 0)),
            pl.BlockSpec((1, npad), lambda: (0, 0)),
            pl.BlockSpec((1, npad), lambda: (0, 0)),
            pl.BlockSpec((1, d), lambda: (0, 0)),
            pl.BlockSpec((1, d), lambda: (0, 0)),
            pl.BlockSpec(memory_space=pltpu.SMEM),
        ],
        out_specs=pl.BlockSpec((1, npad), lambda: (0, 0)),
    )(t, p, rin, W_pred.astype(jnp.float32),
      b_conv1.astype(jnp.float32).reshape(1, d),
      b_pred.astype(jnp.float32).reshape(1, 1))

    return jnp.reshape(out[0, :n], (n, 1))


# speedup vs baseline: 53.9089x; 1.6509x over previous
import functools

import jax
import jax.numpy as jnp
from jax import lax
from jax.experimental import pallas as pl
from jax.experimental.pallas import tpu as pltpu
from jax.experimental.pallas import tpu_sc as plsc

N_CORES = 2
N_SUBCORES = 16
N_WORKERS = N_CORES * N_SUBCORES
LANES = 16

_SC_PARAMS = pltpu.CompilerParams(needs_layout_passes=False)


def _zero_vmem(ref, npad):
    zero = jnp.zeros((LANES,), jnp.float32)

    @plsc.parallel_loop(0, npad, LANES, unroll=8)
    def _(i):
        ref[pl.ds(i, LANES)] = zero


def _publish_reduce_emit(acc, shared, out_hbm, tmps, accr, sem, sid, npad, out_base):
    stripe = npad // N_SUBCORES
    pltpu.sync_copy(acc, shared.at[pl.ds(sid * npad, npad)])
    plsc.subcore_barrier()
    st = sid * stripe
    descs = [
        pltpu.async_copy(
            shared.at[pl.ds(j * npad + st, stripe)],
            tmps.at[pl.ds(j * stripe, stripe)], sem)
        for j in range(N_SUBCORES)
    ]
    for dsc in descs:
        dsc.wait()

    @plsc.parallel_loop(0, stripe, LANES, unroll=4)
    def _(k):
        v = tmps[pl.ds(k, LANES)]
        for j in range(1, N_SUBCORES):
            v = v + tmps[pl.ds(j * stripe + k, LANES)]
        accr[pl.ds(k, LANES)] = v

    pltpu.sync_copy(accr, out_hbm.at[pl.ds(out_base + st, stripe)])
    plsc.subcore_barrier()


def _make_degree_kernel(e, npad):
    ch = e // N_SUBCORES
    stripe = npad // N_SUBCORES
    mesh = plsc.VectorSubcoreMesh(core_axis_name="c", subcore_axis_name="s")

    @functools.partial(
        pl.kernel,
        out_type=jax.ShapeDtypeStruct((2 * npad,), jnp.float32),
        mesh=mesh,
        compiler_params=_SC_PARAMS,
        scratch_types=[
            pltpu.VMEM((ch,), jnp.int32),
            pltpu.VMEM((npad,), jnp.float32),
            pltpu.VMEM((N_SUBCORES * stripe,), jnp.float32),
            pltpu.VMEM((stripe,), jnp.float32),
            pltpu.VMEM_SHARED((N_SUBCORES * npad,), jnp.float32),
            pltpu.SemaphoreType.DMA,
        ],
    )
    def deg_kernel(edges_hbm, out_hbm, idx_v, cnt, tmps, accr, shared, sem):
        cid = lax.axis_index("c")
        sid = lax.axis_index("s")
        pltpu.sync_copy(edges_hbm.at[pl.ds(cid * e + sid * ch, ch)], idx_v)
        _zero_vmem(cnt, npad)
        ones = jnp.full((LANES,), 1.0, jnp.float32)

        @plsc.parallel_loop(0, ch, LANES, unroll=8)
        def _(i):
            ix = idx_v[pl.ds(i, LANES)]
            plsc.addupdate_scatter(cnt, [ix], ones)

        _publish_reduce_emit(cnt, shared, out_hbm, tmps, accr, sem, sid, npad,
                             cid * npad)

    return deg_kernel


def _make_agg4_kernel(e, npad):
    ch = e // N_SUBCORES
    stripe = npad // N_SUBCORES
    mesh = plsc.VectorSubcoreMesh(core_axis_name="c", subcore_axis_name="s")

    @functools.partial(
        pl.kernel,
        out_type=jax.ShapeDtypeStruct((4 * npad,), jnp.float32),
        mesh=mesh,
        compiler_params=_SC_PARAMS,
        scratch_types=[
            pltpu.VMEM((ch,), jnp.int32),
            pltpu.VMEM((ch,), jnp.int32),
            pltpu.VMEM((npad,), jnp.float32),
            pltpu.VMEM((npad,), jnp.float32),
            pltpu.VMEM((npad,), jnp.float32),
            pltpu.VMEM((npad,), jnp.float32),
            pltpu.VMEM((N_SUBCORES * stripe,), jnp.float32),
            pltpu.VMEM((stripe,), jnp.float32),
            pltpu.VMEM_SHARED((N_SUBCORES * npad,), jnp.float32),
            pltpu.SemaphoreType.DMA,
        ],
    )
    def agg_kernel(edges_hbm, tab_hbm, out_hbm, src_v, dst_v, g0, g1, a0, a1,
                   tmps, accr, shared, sem):
        cid = lax.axis_index("c")
        sid = lax.axis_index("s")
        base = sid * ch
        pltpu.sync_copy(edges_hbm.at[pl.ds(base, ch)], src_v)
        pltpu.sync_copy(edges_hbm.at[pl.ds(e + base, ch)], dst_v)
        pltpu.sync_copy(tab_hbm.at[pl.ds((2 * cid) * npad, npad)], g0)
        pltpu.sync_copy(tab_hbm.at[pl.ds((2 * cid + 1) * npad, npad)], g1)
        _zero_vmem(a0, npad)
        _zero_vmem(a1, npad)

        @plsc.parallel_loop(0, ch, LANES, unroll=4)
        def _(i):
            ix_s = src_v[pl.ds(i, LANES)]
            ix_d = dst_v[pl.ds(i, LANES)]
            v0 = plsc.load_gather(g0, [ix_s])
            v1 = plsc.load_gather(g1, [ix_s])
            plsc.addupdate_scatter(a0, [ix_d], v0)
            plsc.addupdate_scatter(a1, [ix_d], v1)

        _publish_reduce_emit(a0, shared, out_hbm, tmps, accr, sem, sid, npad,
                             (2 * cid) * npad)
        _publish_reduce_emit(a1, shared, out_hbm, tmps, accr, sem, sid, npad,
                             (2 * cid + 1) * npad)

    return agg_kernel


def _make_agg1_kernel(e, npad):
    ch = e // N_WORKERS
    stripe = npad // N_SUBCORES
    mesh = plsc.VectorSubcoreMesh(core_axis_name="c", subcore_axis_name="s")

    @functools.partial(
        pl.kernel,
        out_type=jax.ShapeDtypeStruct((2 * npad,), jnp.float32),
        mesh=mesh,
        compiler_params=_SC_PARAMS,
        scratch_types=[
            pltpu.VMEM((ch,), jnp.int32),
            pltpu.VMEM((ch,), jnp.int32),
            pltpu.VMEM((npad,), jnp.float32),
            pltpu.VMEM((npad,), jnp.float32),
            pltpu.VMEM((N_SUBCORES * stripe,), jnp.float32),
            pltpu.VMEM((stripe,), jnp.float32),
            pltpu.VMEM_SHARED((N_SUBCORES * npad,), jnp.float32),
            pltpu.SemaphoreType.DMA,
        ],
    )
    def agg_kernel(edges_hbm, p_hbm, out_hbm, src_v, dst_v, pv, acc,
                   tmps, accr, shared, sem):
        cid = lax.axis_index("c")
        sid = lax.axis_index("s")
        base = cid * (e // 2) + sid * ch
        pltpu.sync_copy(edges_hbm.at[pl.ds(base, ch)], src_v)
        pltpu.sync_copy(edges_hbm.at[pl.ds(e + base, ch)], dst_v)
        pltpu.sync_copy(p_hbm.at[pl.ds(0, npad)], pv)
        _zero_vmem(acc, npad)

        @plsc.parallel_loop(0, ch, LANES, unroll=8)
        def _(i):
            ix_s = src_v[pl.ds(i, LANES)]
            ix_d = dst_v[pl.ds(i, LANES)]
            v = plsc.load_gather(pv, [ix_s])
            plsc.addupdate_scatter(acc, [ix_d], v)

        _publish_reduce_emit(acc, shared, out_hbm, tmps, accr, sem, sid, npad,
                             cid * npad)

    return agg_kernel


def _feat_kernel(deg_ref, w_ref, sig_ref, et_ref, g_ref, rin_ref):
    d_out = deg_ref[0:1, :]
    d_in = deg_ref[1:2, :]
    r_out = lax.rsqrt(d_out + 1.0)
    rin_ref[...] = lax.rsqrt(d_in + 1.0)
    is1 = sig_ref[...] == 1
    s0 = jnp.where(is1, et_ref[1, 0], et_ref[0, 0])
    s1 = jnp.where(is1, et_ref[1, 1], et_ref[0, 1])
    g_ref[0:1, :] = w_ref[...] * r_out
    g_ref[1:2, :] = s0 * r_out
    g_ref[2:3, :] = s1 * r_out
    g_ref[3:4, :] = r_out


def _dense_kernel(agg_ref, g_ref, rin_ref, wlin_ref, wc0_ref,
                  wc1_ref, wp_ref, blin_ref, bc0_ref, p_ref):
    a = (agg_ref[...] + g_ref[...]) * rin_ref[...]
    a3 = a[0:3, :]
    s = a[3:4, :]
    dn = (((1,), (0,)), ((), ()))
    hp = lax.Precision.HIGHEST
    t1 = lax.dot_general(wlin_ref[...], a3, dn,
                         preferred_element_type=jnp.float32, precision=hp)
    t1 = t1 + blin_ref[...] * s
    emb = lax.dot_general(wc0_ref[...], t1, dn,
                          preferred_element_type=jnp.float32, precision=hp)
    emb = emb + bc0_ref[...]
    h = jnp.where(emb >= 0.0, emb, 0.01 * emb)
    t2 = lax.dot_general(wc1_ref[...], h, dn,
                         preferred_element_type=jnp.float32, precision=hp)
    pq = lax.dot_general(wp_ref[...], t2, dn,
                         preferred_element_type=jnp.float32, precision=hp)
    p_ref[...] = pq * g_ref[3:4, :]


def _final_kernel(t_ref, p_ref, rin_ref, wp_ref, bc1_ref, bp_ref, out_ref):
    c = jnp.sum(wp_ref[...] * bc1_ref[...]) + bp_ref[0, 0]
    out_ref[...] = (t_ref[0:1, :] + t_ref[1:2, :] + p_ref[...]) * rin_ref[...] + c


def kernel(significance, weight, edge_index, embed_table, W_lin, b_lin,
           W_conv0, b_conv0, W_conv1, b_conv1, W_pred, b_pred):
    n = significance.shape[0]
    e = edge_index.shape[1]
    d = W_lin.shape[0]
    npad = ((n + 2047) // 2048) * 2048

    edges = edge_index.astype(jnp.int32).reshape(-1)
    wpad = jnp.pad(weight.astype(jnp.float32), (0, npad - n)).reshape(1, npad)
    sigpad = jnp.pad(significance.astype(jnp.int32), (0, npad - n)).reshape(1, npad)

    deg = _make_degree_kernel(e, npad)(edges).reshape(2, npad)

    g, rin = pl.pallas_call(
        _feat_kernel,
        out_shape=(
            jax.ShapeDtypeStruct((4, npad), jnp.float32),
            jax.ShapeDtypeStruct((1, npad), jnp.float32),
        ),
        in_specs=[
            pl.BlockSpec((2, npad), lambda: (0, 0)),
            pl.BlockSpec((1, npad), lambda: (0, 0)),
            pl.BlockSpec((1, npad), lambda: (0, 0)),
            pl.BlockSpec(memory_space=pltpu.SMEM),
        ],
        out_specs=(
            pl.BlockSpec((4, npad), lambda: (0, 0)),
            pl.BlockSpec((1, npad), lambda: (0, 0)),
        ),
    )(deg, wpad, sigpad, embed_table.astype(jnp.float32))

    agg = _make_agg4_kernel(e, npad)(edges, g.reshape(-1)).reshape(4, npad)

    bw = 512
    grid = (npad // bw,)
    p = pl.pallas_call(
        _dense_kernel,
        grid=grid,
        out_shape=jax.ShapeDtypeStruct((1, npad), jnp.float32),
        in_specs=[
            pl.BlockSpec((4, bw), lambda i: (0, i)),
            pl.BlockSpec((4, bw), lambda i: (0, i)),
            pl.BlockSpec((1, bw), lambda i: (0, i)),
            pl.BlockSpec((d, 3), lambda i: (0, 0)),
            pl.BlockSpec((d, d), lambda i: (0, 0)),
            pl.BlockSpec((d, d), lambda i: (0, 0)),
            pl.BlockSpec((1, d), lambda i: (0, 0)),
            pl.BlockSpec((d, 1), lambda i: (0, 0)),
            pl.BlockSpec((d, 1), lambda i: (0, 0)),
        ],
        out_specs=pl.BlockSpec((1, bw), lambda i: (0, i)),
    )(
        agg, g, rin,
        W_lin.astype(jnp.float32), W_conv0.astype(jnp.float32),
        W_conv1.astype(jnp.float32), W_pred.astype(jnp.float32),
        b_lin.astype(jnp.float32).reshape(d, 1),
        b_conv0.astype(jnp.float32).reshape(d, 1),
    )

    t = _make_agg1_kernel(e, npad)(edges, p.reshape(-1)).reshape(2, npad)

    out = pl.pallas_call(
        _final_kernel,
        out_shape=jax.ShapeDtypeStruct((1, npad), jnp.float32),
        in_specs=[
            pl.BlockSpec((2, npad), lambda: (0, 0)),
            pl.BlockSpec((1, npad), lambda: (0, 0)),
            pl.BlockSpec((1, npad), lambda: (0, 0)),
            pl.BlockSpec((1, d), lambda: (0, 0)),
            pl.BlockSpec((1, d), lambda: (0, 0)),
            pl.BlockSpec(memory_space=pltpu.SMEM),
        ],
        out_specs=pl.BlockSpec((1, npad), lambda: (0, 0)),
    )(t, p, rin, W_pred.astype(jnp.float32),
      b_conv1.astype(jnp.float32).reshape(1, d),
      b_pred.astype(jnp.float32).reshape(1, 1))

    return jnp.reshape(out[0, :n], (n, 1))
